# baseline (speedup 1.0000x reference)
import functools
import jax
import jax.numpy as jnp
from jax import lax
from jax.experimental import pallas as pl
from jax.experimental.pallas import tpu as pltpu, tpu_sc as plsc

NW = 32
RX = 32
NBUF = 3


def _make_sc(B, S, D):
    s_per_w = S // NW
    n_chunks = s_per_w // RX
    n_steps = n_chunks * B
    vecs = D // 16
    mesh = plsc.VectorSubcoreMesh(core_axis_name="c", subcore_axis_name="s")

    @functools.partial(
        pl.kernel,
        mesh=mesh,
        out_type=jax.ShapeDtypeStruct((B, S, D), jnp.float32),
        scratch_types=(
            [pltpu.VMEM((RX, D), jnp.float32) for _ in range(2)]
            + [pltpu.VMEM((RX, D), jnp.float32) for _ in range(NBUF)]
            + [pltpu.SemaphoreType.DMA for _ in range(2 + 2 * NBUF)]
        ),
    )
    def k(x_hbm, wpe_hbm, out_hbm, *scratch):
        wbufs = scratch[0:2]
        xbufs = scratch[2:2 + NBUF]
        w_sems = scratch[2 + NBUF:4 + NBUF]
        in_sems = scratch[4 + NBUF:4 + 2 * NBUF]
        out_sems = scratch[4 + 2 * NBUF:4 + 3 * NBUF]
        wid = lax.axis_index("s") * 2 + lax.axis_index("c")
        s0 = wid * s_per_w

        def x_slice(t):
            ci, b = divmod(t, B)
            return (b, pl.ds(s0 + ci * RX, RX))

        def start_in(t):
            return pltpu.async_copy(x_hbm.at[x_slice(t)], xbufs[t % NBUF],
                                    in_sems[t % NBUF])

        w_descs = [None] * n_chunks
        in_descs = [None] * n_steps
        out_descs = [None] * n_steps

        w_descs[0] = pltpu.async_copy(wpe_hbm.at[pl.ds(s0, RX)], wbufs[0],
                                      w_sems[0])
        in_descs[0] = start_in(0)
        in_descs[1] = start_in(1)
        for t in range(n_steps):
            ci, b = divmod(t, B)
            if t >= 1:
                out_descs[t - 1].wait()
            if t + 2 < n_steps:
                in_descs[t + 2] = start_in(t + 2)
            if b == 0:
                if ci + 1 < n_chunks:
                    w_descs[ci + 1] = pltpu.async_copy(
                        wpe_hbm.at[pl.ds(s0 + (ci + 1) * RX, RX)],
                        wbufs[(ci + 1) % 2], w_sems[(ci + 1) % 2])
                w_descs[ci].wait()
            in_descs[t].wait()

            xb = xbufs[t % NBUF]
            wb = wbufs[ci % 2]

            @plsc.parallel_loop(0, RX, 1)
            def _(r):
                @plsc.parallel_loop(0, vecs, 1, unroll=8)
                def _(c):
                    plsc.addupdate(xb.at[r, pl.ds(c * 16, 16)],
                                   wb[r, pl.ds(c * 16, 16)])

            out_descs[t] = pltpu.async_copy(xb, out_hbm.at[x_slice(t)],
                                            out_sems[t % NBUF])
        out_descs[n_steps - 1].wait()

    return k


def kernel(x, wpe):
    B, S, D = x.shape
    return _make_sc(B, S, D)(x, wpe)


# speedup vs baseline: 1.1489x; 1.1489x over previous
import functools
import jax
import jax.numpy as jnp
from jax import lax
from jax.experimental import pallas as pl
from jax.experimental.pallas import tpu as pltpu, tpu_sc as plsc

NW = 32
RX = 8
NBUF = 3
NWBUF = 3


def _make_sc(B, S, D):
    s_per_w = S // NW
    n_chunks = s_per_w // RX
    vecs = D // 16
    mesh = plsc.VectorSubcoreMesh(core_axis_name="c", subcore_axis_name="s")

    @functools.partial(
        pl.kernel,
        mesh=mesh,
        out_type=jax.ShapeDtypeStruct((B, S, D), jnp.float32),
        scratch_types=(
            [pltpu.VMEM((RX, D), jnp.float32) for _ in range(NWBUF)]
            + [pltpu.VMEM((RX, D), jnp.float32) for _ in range(B * NBUF)]
            + [pltpu.SemaphoreType.DMA
               for _ in range(NWBUF + 2 * B * NBUF)]
        ),
    )
    def k(x_hbm, wpe_hbm, out_hbm, *scratch):
        wbufs = scratch[0:NWBUF]
        xbufs = scratch[NWBUF:NWBUF + B * NBUF]
        sems = scratch[NWBUF + B * NBUF:]
        w_sems = sems[0:NWBUF]
        in_sems = sems[NWBUF:NWBUF + B * NBUF]
        out_sems = sems[NWBUF + B * NBUF:]
        wid = lax.axis_index("s") * 2 + lax.axis_index("c")
        s0 = wid * s_per_w

        def rows(ci):
            return pl.ds(s0 + ci * RX, RX)

        def start_ins(ci):
            sl = ci % NBUF
            return [
                pltpu.async_copy(x_hbm.at[(b, rows(ci))],
                                 xbufs[b * NBUF + sl],
                                 in_sems[b * NBUF + sl])
                for b in range(B)
            ]

        def start_w(ci):
            return pltpu.async_copy(wpe_hbm.at[rows(ci)],
                                    wbufs[ci % NWBUF], w_sems[ci % NWBUF])

        w_descs = [None] * n_chunks
        in_descs = [None] * n_chunks
        out_descs = [None] * n_chunks

        w_descs[0] = start_w(0)
        in_descs[0] = start_ins(0)
        w_descs[1] = start_w(1)
        in_descs[1] = start_ins(1)
        for ci in range(n_chunks):
            sl = ci % NBUF
            if ci >= 2:
                for d in out_descs[ci - 2]:
                    d.wait()
            if ci + 2 < n_chunks:
                w_descs[ci + 2] = start_w(ci + 2)
            if ci + 1 < n_chunks and ci + 1 >= 2:
                in_descs[ci + 1] = start_ins(ci + 1)
            w_descs[ci].wait()
            for d in in_descs[ci]:
                d.wait()

            wb = wbufs[ci % NWBUF]
            bbufs = [xbufs[b * NBUF + sl] for b in range(B)]

            @plsc.parallel_loop(0, RX, 1)
            def _(r):
                @plsc.parallel_loop(0, vecs, 1, unroll=4)
                def _(c):
                    wv = wb[r, pl.ds(c * 16, 16)]
                    for b in range(B):
                        plsc.addupdate(bbufs[b].at[r, pl.ds(c * 16, 16)], wv)

            out_descs[ci] = [
                pltpu.async_copy(bbufs[b], out_hbm.at[(b, rows(ci))],
                                 out_sems[b * NBUF + sl])
                for b in range(B)
            ]
        for d in out_descs[n_chunks - 2]:
            d.wait()
        for d in out_descs[n_chunks - 1]:
            d.wait()

    return k


def kernel(x, wpe):
    B, S, D = x.shape
    return _make_sc(B, S, D)(x, wpe)
